# baseline (speedup 1.0000x reference)
import jax
import jax.numpy as jnp
from jax.experimental import pallas as pl
from jax.experimental.pallas import tpu as pltpu


def _copy_body(perm_ref, in_ref, out_ref):
    out_ref[...] = in_ref[...]


def kernel(inp):
    C, H, W = inp.shape
    D = H * W
    perm = jax.random.permutation(jax.random.key(42), C).astype(jnp.int32)
    x = inp.reshape(C, 1, D)
    grid_spec = pltpu.PrefetchScalarGridSpec(
        num_scalar_prefetch=1,
        grid=(C,),
        in_specs=[pl.BlockSpec((1, 1, D), lambda i, perm_ref: (perm_ref[i], 0, 0))],
        out_specs=pl.BlockSpec((1, 1, D), lambda i, perm_ref: (i, 0, 0)),
    )
    out = pl.pallas_call(
        _copy_body,
        grid_spec=grid_spec,
        out_shape=jax.ShapeDtypeStruct((C, 1, D), inp.dtype),
    )(perm, x)
    return out.reshape(C, H, W)


# speedup vs baseline: 1.4594x; 1.4594x over previous
import jax
import jax.numpy as jnp
from jax.experimental import pallas as pl
from jax.experimental.pallas import tpu as pltpu


def _copy_body(perm_ref, in_ref, out_ref):
    out_ref[...] = in_ref[...]


def kernel(inp):
    C, H, W = inp.shape
    D = H * W
    perm = jax.random.permutation(jax.random.key(42), C).astype(jnp.int32)
    del D
    grid_spec = pltpu.PrefetchScalarGridSpec(
        num_scalar_prefetch=1,
        grid=(C,),
        in_specs=[pl.BlockSpec((1, H, W), lambda i, perm_ref: (perm_ref[i], 0, 0))],
        out_specs=pl.BlockSpec((1, H, W), lambda i, perm_ref: (i, 0, 0)),
    )
    return pl.pallas_call(
        _copy_body,
        grid_spec=grid_spec,
        out_shape=jax.ShapeDtypeStruct((C, H, W), inp.dtype),
    )(perm, inp)
